# baseline (speedup 1.0000x reference)
import functools

import jax
import jax.numpy as jnp
import numpy as np
from jax.experimental import pallas as pl
from jax.experimental.pallas import tpu as pltpu


def _conv_sig_kernel(x_ref, w_ref, b_ref, m_ref, o_ref, s_ref, *, cin, cout,
                     k, ho, wo, r0, rows, ipb):
    L = ho * wo
    LX = (k - 1) * wo + L
    b0 = r0 - wo - 1

    chunks = []
    for j in range(ipb):
        s = s_ref.at[j]
        s[:r0, :] = jnp.zeros((r0, cin), jnp.bfloat16)
        s[r0 + L:, :] = jnp.zeros((rows - r0 - L, cin), jnp.bfloat16)
        s[r0:r0 + L, :] = x_ref[j].astype(jnp.bfloat16)

        p0 = s[b0:b0 + LX, :] * m_ref[0, b0:b0 + LX, :]
        p1 = s[b0 + 1:b0 + 1 + LX, :]
        p2 = s[b0 + 2:b0 + 2 + LX, :] * m_ref[1, b0 + 2:b0 + 2 + LX, :]
        chunks.append(jnp.concatenate([p0, p1, p2], axis=1))
    x3 = jnp.concatenate(chunks, axis=0)

    u0 = jnp.dot(x3, w_ref[0], preferred_element_type=jnp.float32)
    u1 = jnp.dot(x3, w_ref[1], preferred_element_type=jnp.float32)
    u2 = jnp.dot(x3, w_ref[2], preferred_element_type=jnp.float32)

    for j in range(ipb):
        o = j * LX
        acc = (u0[o:o + L] + u1[o + wo:o + wo + L]
               + u2[o + 2 * wo:o + 2 * wo + L] + b_ref[...])
        d = 1.0 + jnp.exp(-acc)
        r = pl.reciprocal(d, approx=True)
        o_ref[j] = r * (2.0 - d * r)


def kernel(x_nchw, weight_oihw, bias):
    N, Cin, H, W = x_nchw.shape
    Cout, Cin_w, K, K2 = weight_oihw.shape
    assert Cin == Cin_w and K == K2

    Ho, Wo = H, W
    L = Ho * Wo
    R0 = 64
    ROWS = ((R0 + L + Wo + 2 + 15) // 16) * 16
    IPB = 4
    assert N % IPB == 0

    x_pix = jnp.transpose(x_nchw, (0, 2, 3, 1)).reshape(N, L, Cin)

    w3 = jnp.transpose(weight_oihw, (2, 3, 1, 0)).reshape(K, K * Cin, Cout)
    w3 = w3.astype(jnp.bfloat16)
    b2d = bias.astype(jnp.float32).reshape(1, Cout)

    r = np.arange(ROWS)
    q = r - R0
    in_data = (q >= 0) & (q < L)
    col = q % Wo
    m0 = np.where(in_data & (col == Wo - 1), 0.0, 1.0)
    m2 = np.where(in_data & (col == 0), 0.0, 1.0)
    masks = np.broadcast_to(
        np.stack([m0, m2])[:, :, None], (2, ROWS, Cin)).astype(np.float32)
    masks = jnp.asarray(masks).astype(jnp.bfloat16)

    kernel_fn = functools.partial(_conv_sig_kernel, cin=Cin, cout=Cout, k=K,
                                  ho=Ho, wo=Wo, r0=R0, rows=ROWS, ipb=IPB)

    out = pl.pallas_call(
        kernel_fn,
        out_shape=jax.ShapeDtypeStruct((N, L, Cout), x_nchw.dtype),
        grid=(N // IPB,),
        in_specs=[
            pl.BlockSpec((IPB, L, Cin), lambda n: (n, 0, 0)),
            pl.BlockSpec((K, K * Cin, Cout), lambda n: (0, 0, 0)),
            pl.BlockSpec((1, Cout), lambda n: (0, 0)),
            pl.BlockSpec((2, ROWS, Cin), lambda n: (0, 0, 0)),
        ],
        out_specs=pl.BlockSpec((IPB, L, Cout), lambda n: (n, 0, 0)),
        scratch_shapes=[pltpu.VMEM((IPB, ROWS, Cin), jnp.bfloat16)],
        compiler_params=pltpu.CompilerParams(
            dimension_semantics=("parallel",),
            vmem_limit_bytes=64 * 1024 * 1024,
        ),
    )(x_pix, w3, b2d, masks)
    return jnp.transpose(out.reshape(N, Ho, Wo, Cout), (0, 3, 1, 2))


# speedup vs baseline: 1.0063x; 1.0063x over previous
import functools

import jax
import jax.numpy as jnp
import numpy as np
from jax.experimental import pallas as pl
from jax.experimental.pallas import tpu as pltpu


def _conv_sig_kernel(x_ref, w_ref, b_ref, m_ref, o_ref, s_ref, *, cin, cout,
                     k, ho, wo, r0, rows, ipb):
    L = ho * wo
    LX = (k - 1) * wo + L
    b0 = r0 - wo - 1

    chunks = []
    for j in range(ipb):
        s = s_ref.at[j]
        s[:r0, :] = jnp.zeros((r0, cin), jnp.bfloat16)
        s[r0 + L:, :] = jnp.zeros((rows - r0 - L, cin), jnp.bfloat16)
        s[r0:r0 + L, :] = x_ref[j].astype(jnp.bfloat16)

        p0 = s[b0:b0 + LX, :] * m_ref[0, b0:b0 + LX, :]
        p1 = s[b0 + 1:b0 + 1 + LX, :]
        p2 = s[b0 + 2:b0 + 2 + LX, :] * m_ref[1, b0 + 2:b0 + 2 + LX, :]
        chunks.append(jnp.concatenate([p0, p1, p2], axis=1))
    x3 = jnp.concatenate(chunks, axis=0)

    u0 = jnp.dot(x3, w_ref[0], preferred_element_type=jnp.float32)
    u1 = jnp.dot(x3, w_ref[1], preferred_element_type=jnp.float32)
    u2 = jnp.dot(x3, w_ref[2], preferred_element_type=jnp.float32)

    for j in range(ipb):
        o = j * LX
        acc = (u0[o:o + L] + u1[o + wo:o + wo + L]
               + u2[o + 2 * wo:o + 2 * wo + L] + b_ref[...])
        d = 1.0 + jnp.exp(-acc)
        r = pl.reciprocal(d, approx=True)
        o_ref[j] = r * (2.0 - d * r)


def kernel(x_nchw, weight_oihw, bias):
    N, Cin, H, W = x_nchw.shape
    Cout, Cin_w, K, K2 = weight_oihw.shape
    assert Cin == Cin_w and K == K2

    Ho, Wo = H, W
    L = Ho * Wo
    R0 = 64
    ROWS = ((R0 + L + Wo + 2 + 15) // 16) * 16
    IPB = 2
    assert N % IPB == 0

    x_pix = jnp.transpose(x_nchw, (0, 2, 3, 1)).reshape(N, L, Cin)

    w3 = jnp.transpose(weight_oihw, (2, 3, 1, 0)).reshape(K, K * Cin, Cout)
    w3 = w3.astype(jnp.bfloat16)
    b2d = bias.astype(jnp.float32).reshape(1, Cout)

    r = np.arange(ROWS)
    q = r - R0
    in_data = (q >= 0) & (q < L)
    col = q % Wo
    m0 = np.where(in_data & (col == Wo - 1), 0.0, 1.0)
    m2 = np.where(in_data & (col == 0), 0.0, 1.0)
    masks = np.broadcast_to(
        np.stack([m0, m2])[:, :, None], (2, ROWS, Cin)).astype(np.float32)
    masks = jnp.asarray(masks).astype(jnp.bfloat16)

    kernel_fn = functools.partial(_conv_sig_kernel, cin=Cin, cout=Cout, k=K,
                                  ho=Ho, wo=Wo, r0=R0, rows=ROWS, ipb=IPB)

    out = pl.pallas_call(
        kernel_fn,
        out_shape=jax.ShapeDtypeStruct((N, L, Cout), x_nchw.dtype),
        grid=(N // IPB,),
        in_specs=[
            pl.BlockSpec((IPB, L, Cin), lambda n: (n, 0, 0)),
            pl.BlockSpec((K, K * Cin, Cout), lambda n: (0, 0, 0)),
            pl.BlockSpec((1, Cout), lambda n: (0, 0)),
            pl.BlockSpec((2, ROWS, Cin), lambda n: (0, 0, 0)),
        ],
        out_specs=pl.BlockSpec((IPB, L, Cout), lambda n: (n, 0, 0)),
        scratch_shapes=[pltpu.VMEM((IPB, ROWS, Cin), jnp.bfloat16)],
        compiler_params=pltpu.CompilerParams(
            dimension_semantics=("parallel",),
            vmem_limit_bytes=64 * 1024 * 1024,
        ),
    )(x_pix, w3, b2d, masks)
    return jnp.transpose(out.reshape(N, Ho, Wo, Cout), (0, 3, 1, 2))


# speedup vs baseline: 1.0088x; 1.0026x over previous
import functools

import jax
import jax.numpy as jnp
import numpy as np
from jax.experimental import pallas as pl
from jax.experimental.pallas import tpu as pltpu


def _conv_sig_kernel(x_ref, w_ref, b_ref, m_ref, o_ref, s_ref, *, cin, cout,
                     k, ho, wo, r0, rows, ipb):
    L = ho * wo
    LX = (k - 1) * wo + L
    b0 = r0 - wo - 1

    w3 = [jnp.concatenate(
        [w_ref[kh * k + kw].T for kw in range(k)],
        axis=0).astype(jnp.bfloat16) for kh in range(k)]

    chunks = []
    for j in range(ipb):
        s = s_ref.at[j]
        s[:r0, :] = jnp.zeros((r0, cin), jnp.bfloat16)
        s[r0 + L:, :] = jnp.zeros((rows - r0 - L, cin), jnp.bfloat16)
        s[r0:r0 + L, :] = x_ref[j].astype(jnp.bfloat16)

        p0 = s[b0:b0 + LX, :] * m_ref[0, b0:b0 + LX, :]
        p1 = s[b0 + 1:b0 + 1 + LX, :]
        p2 = s[b0 + 2:b0 + 2 + LX, :] * m_ref[1, b0 + 2:b0 + 2 + LX, :]
        chunks.append(jnp.concatenate([p0, p1, p2], axis=1))
    x3 = jnp.concatenate(chunks, axis=0)

    u0 = jnp.dot(x3, w3[0], preferred_element_type=jnp.float32)
    u1 = jnp.dot(x3, w3[1], preferred_element_type=jnp.float32)
    u2 = jnp.dot(x3, w3[2], preferred_element_type=jnp.float32)

    for j in range(ipb):
        o = j * LX
        acc = (u0[o:o + L] + u1[o + wo:o + wo + L]
               + u2[o + 2 * wo:o + 2 * wo + L] + b_ref[...])
        d = 1.0 + jnp.exp(-acc)
        r = pl.reciprocal(d, approx=True)
        o_ref[j] = r * (2.0 - d * r)


def kernel(x_nchw, weight_oihw, bias):
    N, Cin, H, W = x_nchw.shape
    Cout, Cin_w, K, K2 = weight_oihw.shape
    assert Cin == Cin_w and K == K2

    Ho, Wo = H, W
    L = Ho * Wo
    R0 = 64
    ROWS = ((R0 + L + Wo + 2 + 15) // 16) * 16
    IPB = 2
    assert N % IPB == 0

    x_pix = jnp.transpose(x_nchw, (0, 2, 3, 1)).reshape(N, L, Cin)

    w9 = jnp.transpose(weight_oihw, (2, 3, 0, 1)).reshape(K * K, Cout, Cin)
    b2d = bias.astype(jnp.float32).reshape(1, Cout)

    r = np.arange(ROWS)
    q = r - R0
    in_data = (q >= 0) & (q < L)
    col = q % Wo
    m0 = np.where(in_data & (col == Wo - 1), 0.0, 1.0)
    m2 = np.where(in_data & (col == 0), 0.0, 1.0)
    masks = np.broadcast_to(
        np.stack([m0, m2])[:, :, None], (2, ROWS, Cin)).astype(np.float32)
    masks = jnp.asarray(masks).astype(jnp.bfloat16)

    kernel_fn = functools.partial(_conv_sig_kernel, cin=Cin, cout=Cout, k=K,
                                  ho=Ho, wo=Wo, r0=R0, rows=ROWS, ipb=IPB)

    out = pl.pallas_call(
        kernel_fn,
        out_shape=jax.ShapeDtypeStruct((N, L, Cout), x_nchw.dtype),
        grid=(N // IPB,),
        in_specs=[
            pl.BlockSpec((IPB, L, Cin), lambda n: (n, 0, 0)),
            pl.BlockSpec((K * K, Cout, Cin), lambda n: (0, 0, 0)),
            pl.BlockSpec((1, Cout), lambda n: (0, 0)),
            pl.BlockSpec((2, ROWS, Cin), lambda n: (0, 0, 0)),
        ],
        out_specs=pl.BlockSpec((IPB, L, Cout), lambda n: (n, 0, 0)),
        scratch_shapes=[pltpu.VMEM((IPB, ROWS, Cin), jnp.bfloat16)],
        compiler_params=pltpu.CompilerParams(
            dimension_semantics=("parallel",),
            vmem_limit_bytes=64 * 1024 * 1024,
        ),
    )(x_pix, w9, b2d, masks)
    return jnp.transpose(out.reshape(N, Ho, Wo, Cout), (0, 3, 1, 2))
